# Initial kernel scaffold; baseline (speedup 1.0000x reference)
import functools

import jax
import jax.numpy as jnp
from jax import lax
from jax.experimental import pallas as pl
from jax.experimental.pallas import tpu as pltpu
from jax.experimental.pallas import tpu_sc as plsc

NEG = -1e30
LPAD = 256
TCHUNK = 200
RESCALE = 4
SC_ROWS = 50


def _sc_gather_call(nnet, labels_pad):
    B, T, C = nnet.shape
    Lp = labels_pad.shape[1]
    info = plsc.get_sparse_core_info()
    nw = info.num_cores * info.num_subcores
    halves = nw // B
    t_per_w = T // halves
    n_chunks = t_per_w // SC_ROWS
    mesh = plsc.VectorSubcoreMesh(core_axis_name="c", subcore_axis_name="s")

    @functools.partial(
        pl.kernel,
        out_type=jax.ShapeDtypeStruct((B, T, LPAD), jnp.float32),
        mesh=mesh,
        scratch_types=[
            pltpu.VMEM((Lp,), jnp.int32),
            pltpu.VMEM((SC_ROWS, C), jnp.float32),
            pltpu.VMEM((SC_ROWS, LPAD), jnp.float32),
        ],
    )
    def gather_kernel(nnet_hbm, lab_hbm, out_hbm, lab_v, x_v, o_v):
        wid = lax.axis_index("s") * info.num_cores + lax.axis_index("c")
        b = wid // halves
        half = wid % halves
        pltpu.sync_copy(lab_hbm.at[b], lab_v)
        for r in range(SC_ROWS):
            for j in range(Lp // 16, LPAD // 16):
                o_v[r, pl.ds(j * 16, 16)] = jnp.zeros((16,), jnp.float32)

        def do_chunk(g, carry):
            t0 = half * t_per_w + g * SC_ROWS
            pltpu.sync_copy(nnet_hbm.at[b, pl.ds(t0, SC_ROWS)], x_v)

            def do_row(r, carry2):
                ridx = jnp.full((16,), r, jnp.int32)
                for j in range(Lp // 16):
                    cidx = lab_v[pl.ds(j * 16, 16)]
                    o_v[r, pl.ds(j * 16, 16)] = plsc.load_gather(
                        x_v, [ridx, cidx])
                return carry2

            lax.fori_loop(0, SC_ROWS, do_row, 0, unroll=False)
            pltpu.sync_copy(o_v, out_hbm.at[b, pl.ds(t0, SC_ROWS)])
            return carry

        lax.fori_loop(0, n_chunks, do_chunk, 0, unroll=False)

    return gather_kernel(nnet, labels_pad)


def _fb_body(L, nf_ref, nnet_ref, emis_ref, P_ref,
             score_ref, tf_ref, af_ref, a_ref, p_ref, m_ref):
    k = pl.program_id(0)
    nk = pl.num_programs(0)
    Bb, Tc, Cc = nnet_ref.shape

    @pl.when(k == 0)
    def _init():
        lane = lax.broadcasted_iota(jnp.int32, (Bb, LPAD), 1)
        a_ref[...] = jnp.where(lane == 0, 0.0, NEG).astype(jnp.float32)
        p_ref[...] = jnp.ones((Bb, Cc), jnp.float32)
        m_ref[...] = jnp.full((Bb, 128), -jnp.log(float(Cc)), jnp.float32)

    expP = jnp.exp(P_ref[...])

    def one_step(t, a, p, m):
        e_t = emis_ref[:, t, :]
        mv = jnp.concatenate(
            [jnp.full((Bb, 1), NEG, jnp.float32), a[:, :-1]], axis=1)
        hi = jnp.maximum(a, mv)
        lo = jnp.minimum(a, mv)
        a = hi + jnp.log1p(jnp.exp(lo - hi)) + e_t
        x_t = nnet_ref[:, t, :]
        mx = jnp.max(x_t, axis=1, keepdims=True)
        s_t = jnp.exp(x_t - mx)
        p = lax.dot_general(p, expP, (((1,), (0,)), ((), ())),
                            preferred_element_type=jnp.float32) * s_t
        m = m + mx
        return a, p, m

    def block(i, carry):
        a, p, m = carry
        for j in range(RESCALE):
            a, p, m = one_step(i * RESCALE + j, a, p, m)
        r = jnp.max(p, axis=1, keepdims=True)
        p = p * (1.0 / r)
        m = m + jnp.log(r)
        return a, p, m

    carry0 = (a_ref[...], p_ref[...], m_ref[:, 0:1])
    a, p, m = lax.fori_loop(0, Tc // RESCALE, block, carry0, unroll=False)
    a_ref[...] = a
    p_ref[...] = p
    m_ref[...] = jnp.broadcast_to(m, (Bb, 128))

    @pl.when(k == nk - 1)
    def _fin():
        num = a[:, L - 1:L]
        den = jnp.log(jnp.sum(p, axis=1, keepdims=True)) + m
        sc = num - den
        nf = nf_ref[...]
        okm = jnp.isfinite(sc) & (sc > NEG / 2)
        score_ref[0, 0] = jnp.sum(jnp.where(okm, sc, 0.0))
        tf_ref[0, 0] = jnp.sum(jnp.where(okm, nf, 0))
        af_ref[0, 0] = jnp.sum(nf)


def _fb_call(L, nf2, nnet, emis, P, interpret=False):
    B, T, C = nnet.shape
    nk = T // TCHUNK
    return pl.pallas_call(
        functools.partial(_fb_body, L),
        grid=(nk,),
        in_specs=[
            pl.BlockSpec((B, 1), lambda k: (0, 0)),
            pl.BlockSpec((B, TCHUNK, C), lambda k: (0, k, 0)),
            pl.BlockSpec((B, TCHUNK, LPAD), lambda k: (0, k, 0)),
            pl.BlockSpec((C, C), lambda k: (0, 0)),
        ],
        out_specs=[
            pl.BlockSpec(memory_space=pltpu.SMEM),
            pl.BlockSpec(memory_space=pltpu.SMEM),
            pl.BlockSpec(memory_space=pltpu.SMEM),
        ],
        out_shape=[
            jax.ShapeDtypeStruct((1, 1), jnp.float32),
            jax.ShapeDtypeStruct((1, 1), jnp.int32),
            jax.ShapeDtypeStruct((1, 1), jnp.int32),
        ],
        scratch_shapes=[
            pltpu.VMEM((B, LPAD), jnp.float32),
            pltpu.VMEM((B, C), jnp.float32),
            pltpu.VMEM((B, 128), jnp.float32),
        ],
        interpret=interpret,
    )(nf2, nnet, emis, P)


def kernel(nnet_output, labels, supervision_segments, P):
    B, T, C = nnet_output.shape
    L = labels.shape[1]
    lp = (-L) % 16
    labels_pad = jnp.pad(labels, ((0, 0), (0, lp)))
    emis = _sc_gather_call(nnet_output, labels_pad)
    nf2 = supervision_segments[:, 2:3]
    score, tf, af = _fb_call(L, nf2, nnet_output, emis, P)
    return score[0, 0], tf[0, 0], af[0, 0]


# speedup vs baseline: 26.0385x; 26.0385x over previous
import functools

import jax
import jax.numpy as jnp
from jax import lax
from jax.experimental import pallas as pl
from jax.experimental.pallas import tpu as pltpu
from jax.experimental.pallas import tpu_sc as plsc

NEG = -1e30
LPAD = 256
TCHUNK = 200
RESCALE = 4
SC_ROWS = 40


def _sc_gather_call(nnet, labels_pad):
    B, T, C = nnet.shape
    Lp = labels_pad.shape[1]
    info = plsc.get_sparse_core_info()
    nw = info.num_cores * info.num_subcores
    halves = nw // B
    t_per_w = T // halves
    n_chunks = t_per_w // SC_ROWS
    mesh = plsc.VectorSubcoreMesh(core_axis_name="c", subcore_axis_name="s")

    @functools.partial(
        pl.kernel,
        out_type=jax.ShapeDtypeStruct((B, T, LPAD), jnp.float32),
        mesh=mesh,
        scratch_types=[
            pltpu.VMEM((Lp,), jnp.int32),
            pltpu.VMEM((SC_ROWS, C), jnp.float32),
            pltpu.VMEM((SC_ROWS, LPAD), jnp.float32),
        ],
        compiler_params=pltpu.CompilerParams(needs_layout_passes=False),
    )
    def gather_kernel(nnet_hbm, lab_hbm, out_hbm, lab_v, x_v, o_v):
        wid = lax.axis_index("s") * info.num_cores + lax.axis_index("c")
        b = wid // halves
        half = wid % halves
        pltpu.sync_copy(lab_hbm.at[b], lab_v)
        for r in range(SC_ROWS):
            for j in range(Lp // 16, LPAD // 16):
                o_v[r, pl.ds(j * 16, 16)] = jnp.zeros((16,), jnp.float32)

        def do_chunk(g, carry):
            t0 = half * t_per_w + g * SC_ROWS
            pltpu.sync_copy(nnet_hbm.at[b, pl.ds(t0, SC_ROWS)], x_v)

            def do_row(r, carry2):
                ridx = jnp.full((16,), r, jnp.int32)
                for j in range(Lp // 16):
                    cidx = lab_v[pl.ds(j * 16, 16)]
                    o_v[r, pl.ds(j * 16, 16)] = plsc.load_gather(
                        x_v, [ridx, cidx])
                return carry2

            lax.fori_loop(0, SC_ROWS, do_row, 0, unroll=False)
            pltpu.sync_copy(o_v, out_hbm.at[b, pl.ds(t0, SC_ROWS)])
            return carry

        lax.fori_loop(0, n_chunks, do_chunk, 0, unroll=False)

    return gather_kernel(nnet, labels_pad)


def _fb_body(L, nf_ref, nnet_ref, emis_ref, P_ref,
             score_ref, tf_ref, af_ref, a_ref, p_ref, m_ref):
    k = pl.program_id(0)
    nk = pl.num_programs(0)
    Bb, Tc, Cc = nnet_ref.shape

    @pl.when(k == 0)
    def _init():
        lane = lax.broadcasted_iota(jnp.int32, (Bb, LPAD), 1)
        a_ref[...] = jnp.where(lane == 0, 0.0, NEG).astype(jnp.float32)
        p_ref[...] = jnp.ones((Bb, Cc), jnp.float32)
        m_ref[...] = jnp.full((Bb, 128), -jnp.log(float(Cc)), jnp.float32)

    expP = jnp.exp(P_ref[...])

    def one_step(t, a, p, m):
        e_t = emis_ref[:, t, :]
        mv = jnp.concatenate(
            [jnp.full((Bb, 1), NEG, jnp.float32), a[:, :-1]], axis=1)
        hi = jnp.maximum(a, mv)
        lo = jnp.minimum(a, mv)
        a = hi + jnp.log1p(jnp.exp(lo - hi)) + e_t
        x_t = nnet_ref[:, t, :]
        mx = jnp.max(x_t, axis=1, keepdims=True)
        s_t = jnp.exp(x_t - mx)
        p = lax.dot_general(p, expP, (((1,), (0,)), ((), ())),
                            preferred_element_type=jnp.float32) * s_t
        m = m + mx
        return a, p, m

    def block(i, carry):
        a, p, m = carry
        for j in range(RESCALE):
            a, p, m = one_step(i * RESCALE + j, a, p, m)
        r = jnp.max(p, axis=1, keepdims=True)
        p = p * (1.0 / r)
        m = m + jnp.log(r)
        return a, p, m

    carry0 = (a_ref[...], p_ref[...], m_ref[:, 0:1])
    a, p, m = lax.fori_loop(0, Tc // RESCALE, block, carry0, unroll=False)
    a_ref[...] = a
    p_ref[...] = p
    m_ref[...] = jnp.broadcast_to(m, (Bb, 128))

    @pl.when(k == nk - 1)
    def _fin():
        num = a[:, L - 1:L]
        den = jnp.log(jnp.sum(p, axis=1, keepdims=True)) + m
        sc = num - den
        nf = nf_ref[...]
        okm = jnp.isfinite(sc) & (sc > NEG / 2)
        score_ref[0, 0] = jnp.sum(jnp.where(okm, sc, 0.0))
        tf_ref[0, 0] = jnp.sum(jnp.where(okm, nf, 0))
        af_ref[0, 0] = jnp.sum(nf)


def _fb_call(L, nf2, nnet, emis, P):
    B, T, C = nnet.shape
    nk = T // TCHUNK
    return pl.pallas_call(
        functools.partial(_fb_body, L),
        grid=(nk,),
        in_specs=[
            pl.BlockSpec((B, 1), lambda k: (0, 0)),
            pl.BlockSpec((B, TCHUNK, C), lambda k: (0, k, 0)),
            pl.BlockSpec((B, TCHUNK, LPAD), lambda k: (0, k, 0)),
            pl.BlockSpec((C, C), lambda k: (0, 0)),
        ],
        out_specs=[
            pl.BlockSpec(memory_space=pltpu.SMEM),
            pl.BlockSpec(memory_space=pltpu.SMEM),
            pl.BlockSpec(memory_space=pltpu.SMEM),
        ],
        out_shape=[
            jax.ShapeDtypeStruct((1, 1), jnp.float32),
            jax.ShapeDtypeStruct((1, 1), jnp.int32),
            jax.ShapeDtypeStruct((1, 1), jnp.int32),
        ],
        scratch_shapes=[
            pltpu.VMEM((B, LPAD), jnp.float32),
            pltpu.VMEM((B, C), jnp.float32),
            pltpu.VMEM((B, 128), jnp.float32),
        ],
    )(nf2, nnet, emis, P)


def kernel(nnet_output, labels, supervision_segments, P):
    B, T, C = nnet_output.shape
    L = labels.shape[1]
    lp = (-L) % 16
    labels_pad = jnp.pad(labels, ((0, 0), (0, lp)))
    emis = _sc_gather_call(nnet_output, labels_pad)
    nf2 = supervision_segments[:, 2:3]
    score, tf, af = _fb_call(L, nf2, nnet_output, emis, P)
    return score[0, 0], tf[0, 0], af[0, 0]
